# baseline (speedup 1.0000x reference)
import functools

import jax
import jax.numpy as jnp
from jax import lax
from jax.experimental import pallas as pl
from jax.experimental.pallas import tpu as pltpu
from jax.experimental.pallas import tpu_sc as plsc

B = 4096
D = 128
K = 8

NC = 2
NS = 16
NW = NC * NS
BPW = B // NW

BM = 512


_sc_mesh = plsc.VectorSubcoreMesh(core_axis_name="c", subcore_axis_name="s")


@functools.partial(
    pl.kernel,
    mesh=_sc_mesh,
    out_type=(
        jax.ShapeDtypeStruct((B, D), jnp.float32),
        jax.ShapeDtypeStruct((B, D), jnp.float32),
    ),
    scratch_types=[
        pltpu.VMEM((BPW,), jnp.int32),
        pltpu.VMEM((BPW,), jnp.int32),
        pltpu.VMEM((BPW, D), jnp.float32),
        pltpu.VMEM((BPW, D), jnp.float32),
        pltpu.SemaphoreType.DMA,
        pltpu.SemaphoreType.DMA,
        pltpu.SemaphoreType.DMA,
        pltpu.SemaphoreType.DMA,
    ],
)
def _sc_gather(uid_hbm, iid_hbm, utab_hbm, itab_hbm, uout_hbm, iout_hbm,
               uidx_v, iidx_v, urows_v, irows_v, usem, isem, wusem, wisem):
    wid = lax.axis_index("s") * NC + lax.axis_index("c")
    base = wid * BPW
    lu = pltpu.async_copy(uid_hbm.at[pl.ds(base, BPW)], uidx_v, usem)
    li = pltpu.async_copy(iid_hbm.at[pl.ds(base, BPW)], iidx_v, isem)
    lu.wait()
    cu = pltpu.async_copy(utab_hbm.at[uidx_v], urows_v, usem)
    li.wait()
    ci = pltpu.async_copy(itab_hbm.at[iidx_v], irows_v, isem)
    cu.wait()
    wu = pltpu.async_copy(urows_v, uout_hbm.at[pl.ds(base, BPW)], wusem)
    ci.wait()
    wi = pltpu.async_copy(irows_v, iout_hbm.at[pl.ds(base, BPW)], wisem)
    wu.wait()
    wi.wait()


def _tc_body(uemb_hbm, iemb_hbm, w_ref, b_ref, iv_ref, out_ref,
             uemb_v, iemb_v, usem, isem):
    nblk = B // BM
    cu = pltpu.make_async_copy(uemb_hbm, uemb_v, usem)
    cu.start()
    c0 = pltpu.make_async_copy(iemb_hbm.at[pl.ds(0, BM)], iemb_v.at[0],
                               isem.at[0])
    c0.start()
    cu.wait()
    logits = jnp.dot(uemb_v[...], w_ref[...],
                     preferred_element_type=jnp.float32) + b_ref[...]
    m = jnp.max(logits, axis=-1, keepdims=True)
    e = jnp.exp(logits - m)
    scores = e / jnp.sum(e, axis=-1, keepdims=True)
    ui = jnp.dot(scores, iv_ref[...], preferred_element_type=jnp.float32)
    ui_bf = ui.astype(jnp.bfloat16)
    for i in range(nblk):
        pltpu.make_async_copy(iemb_hbm.at[pl.ds(i * BM, BM)],
                              iemb_v.at[i % 2], isem.at[i % 2]).wait()
        if i + 1 < nblk:
            pltpu.make_async_copy(iemb_hbm.at[pl.ds((i + 1) * BM, BM)],
                                  iemb_v.at[(i + 1) % 2],
                                  isem.at[(i + 1) % 2]).start()
        sims = lax.dot_general(
            ui_bf, iemb_v[i % 2].astype(jnp.bfloat16),
            (((1,), (1,)), ((), ())),
            preferred_element_type=jnp.float32)
        out_ref[pl.ds(i * BM, BM)] = jnp.max(sims, axis=0)


def kernel(user_ids, item_ids, user_table, item_table, interest_vectors,
           attn_W, attn_b):
    uids = user_ids.astype(jnp.int32)
    iids = item_ids.astype(jnp.int32)
    user_emb, item_emb = _sc_gather(uids, iids, user_table, item_table)
    return pl.pallas_call(
        _tc_body,
        in_specs=[
            pl.BlockSpec(memory_space=pl.ANY),
            pl.BlockSpec(memory_space=pl.ANY),
            pl.BlockSpec((D, K), lambda: (0, 0)),
            pl.BlockSpec((1, K), lambda: (0, 0)),
            pl.BlockSpec((K, D), lambda: (0, 0)),
        ],
        out_specs=pl.BlockSpec((B,), lambda: (0,)),
        scratch_shapes=[
            pltpu.VMEM((B, D), jnp.float32),
            pltpu.VMEM((2, BM, D), jnp.float32),
            pltpu.SemaphoreType.DMA,
            pltpu.SemaphoreType.DMA((2,)),
        ],
        out_shape=jax.ShapeDtypeStruct((B,), jnp.float32),
    )(user_emb, item_emb, attn_W, attn_b.reshape(1, K), interest_vectors)


# speedup vs baseline: 1.0511x; 1.0511x over previous
import functools

import jax
import jax.numpy as jnp
from jax import lax
from jax.experimental import pallas as pl
from jax.experimental.pallas import tpu as pltpu
from jax.experimental.pallas import tpu_sc as plsc

B = 4096
D = 128
K = 8

NC = 2
NS = 16
NW = NC * NS
BPW = B // NW

BM = 512


_sc_mesh = plsc.VectorSubcoreMesh(core_axis_name="c", subcore_axis_name="s")


@functools.partial(
    pl.kernel,
    mesh=_sc_mesh,
    out_type=jax.ShapeDtypeStruct((2 * B, D), jnp.float32),
    scratch_types=[
        pltpu.VMEM((BPW,), jnp.int32),
        pltpu.VMEM((BPW,), jnp.int32),
        pltpu.VMEM((BPW, D), jnp.float32),
        pltpu.VMEM((BPW, D), jnp.float32),
        pltpu.SemaphoreType.DMA,
        pltpu.SemaphoreType.DMA,
        pltpu.SemaphoreType.DMA,
        pltpu.SemaphoreType.DMA,
    ],
)
def _sc_gather(ids_hbm, utab_hbm, itab_hbm, emb_hbm,
               uidx_v, iidx_v, urows_v, irows_v, usem, isem, wusem, wisem):
    wid = lax.axis_index("s") * NC + lax.axis_index("c")
    base = wid * BPW
    lu = pltpu.async_copy(ids_hbm.at[pl.ds(base, BPW)], uidx_v, usem)
    li = pltpu.async_copy(ids_hbm.at[pl.ds(B + base, BPW)], iidx_v, isem)
    lu.wait()
    cu = pltpu.async_copy(utab_hbm.at[uidx_v], urows_v, usem)
    li.wait()
    ci = pltpu.async_copy(itab_hbm.at[iidx_v], irows_v, isem)
    cu.wait()
    wu = pltpu.async_copy(urows_v, emb_hbm.at[pl.ds(base, BPW)], wusem)
    ci.wait()
    wi = pltpu.async_copy(irows_v, emb_hbm.at[pl.ds(B + base, BPW)], wisem)
    wu.wait()
    wi.wait()


def _tc_body(emb_ref, w_ref, b_ref, iv_ref, out_ref):
    logits = jnp.dot(emb_ref[0:B, :], w_ref[...],
                     preferred_element_type=jnp.float32) + b_ref[...]
    m = jnp.max(logits, axis=-1, keepdims=True)
    e = jnp.exp(logits - m)
    scores = e / jnp.sum(e, axis=-1, keepdims=True)
    ui = jnp.dot(scores, iv_ref[...], preferred_element_type=jnp.float32)
    ui_bf = ui.astype(jnp.bfloat16)
    for i in range(B // BM):
        sims = lax.dot_general(
            ui_bf, emb_ref[pl.ds(B + i * BM, BM), :].astype(jnp.bfloat16),
            (((1,), (1,)), ((), ())),
            preferred_element_type=jnp.float32)
        out_ref[pl.ds(i * BM, BM)] = jnp.max(sims, axis=0)


def kernel(user_ids, item_ids, user_table, item_table, interest_vectors,
           attn_W, attn_b):
    ids = jnp.concatenate([user_ids.astype(jnp.int32),
                           item_ids.astype(jnp.int32)])
    emb = _sc_gather(ids, user_table, item_table)
    return pl.pallas_call(
        _tc_body,
        out_shape=jax.ShapeDtypeStruct((B,), jnp.float32),
    )(emb, attn_W, attn_b.reshape(1, K), interest_vectors)
